# baseline (speedup 1.0000x reference)
import functools

import jax
import jax.numpy as jnp
from jax import lax
from jax.experimental import pallas as pl
from jax.experimental.pallas import tpu as pltpu
from jax.experimental.pallas import tpu_sc as plsc

PC_RANGE = (0.0, -40.0, -3.0, 70.4, 40.0, 1.0)
PILLAR = 0.1
H = 800
W_GRID = 704
C_OUT = 32

NBIN = 1600
TRASH = NBIN
NB_PAD = 1664
T = 32
PT = 9472
N_PAD = PT * T
RPT = 50
NGRP = 25
ROW_W = W_GRID * C_OUT
SLAB_W = 2 * ROW_W
NR = 310272

_BLK = 2048

_SC_PARAMS = pltpu.CompilerParams(needs_layout_passes=False,
                                  use_tc_tiling_on_sc=False)


def _pfe_body(pts_ref, bo_ref, wm_ref, bb_ref, h_ref, slot_ref):
    pts = pts_ref[...]
    x = pts[:, 0]
    y = pts[:, 1]
    cx = jnp.floor((x - PC_RANGE[0]) / PILLAR).astype(jnp.int32)
    cy = jnp.floor((y - PC_RANGE[1]) / PILLAR).astype(jnp.int32)
    mask = (cx >= 0) & (cx < W_GRID) & (cy >= 0) & (cy < H)
    cxc = jnp.clip(cx, 0, W_GRID - 1)
    cyc = jnp.clip(cy, 0, H - 1)
    center_x = (cxc.astype(jnp.float32) + 0.5) * PILLAR + PC_RANGE[0]
    center_y = (cyc.astype(jnp.float32) + 0.5) * PILLAR + PC_RANGE[1]
    feats = jnp.concatenate(
        [pts[:, :5], (x - center_x)[:, None], (y - center_y)[:, None]], axis=1
    )
    h = jnp.dot(feats, wm_ref[...], preferred_element_type=jnp.float32) + bb_ref[...]
    h_ref[...] = jnp.maximum(h, 0.0)
    slot = (bo_ref[...] + cyc) * 1024 + cxc
    slot_ref[...] = jnp.where(mask, slot, jnp.int32(TRASH * 1024))


def _stage_a(pts_flat, bo, wm, bb):
    return pl.pallas_call(
        _pfe_body,
        grid=(N_PAD // _BLK,),
        in_specs=[
            pl.BlockSpec((_BLK, 8), lambda i: (i, 0)),
            pl.BlockSpec((_BLK,), lambda i: (i,)),
            pl.BlockSpec((7, C_OUT), lambda i: (0, 0)),
            pl.BlockSpec((1, C_OUT), lambda i: (0, 0)),
        ],
        out_specs=[
            pl.BlockSpec((_BLK, C_OUT), lambda i: (i, 0)),
            pl.BlockSpec((_BLK,), lambda i: (i,)),
        ],
        out_shape=[
            jax.ShapeDtypeStruct((N_PAD, C_OUT), jnp.float32),
            jax.ShapeDtypeStruct((N_PAD,), jnp.int32),
        ],
    )(pts_flat, bo, wm, bb)


def _mesh():
    return plsc.VectorSubcoreMesh(core_axis_name="c", subcore_axis_name="s")


def _wid():
    return lax.axis_index("s") * 2 + lax.axis_index("c")


def _zero_i32(ref, nvec):
    def body(i, _):
        ref[pl.ds(i * 16, 16)] = jnp.zeros((16,), jnp.int32)
        return 0
    lax.fori_loop(0, nvec, body, 0)


def _sortmeta(vbin, ktmp, iota16):
    k_srt, perm = plsc.sort_key_val(vbin, iota16)
    ktmp[...] = k_srt
    prev = plsc.load_gather(ktmp, [jnp.maximum(iota16 - 1, 0)])
    newseg = (k_srt != prev) | (iota16 == 0)
    segstart = plsc.cummax(jnp.where(newseg, iota16, 0))
    rank = iota16 - segstart
    nxt = plsc.load_gather(ktmp, [jnp.minimum(iota16 + 1, 15)])
    last = (k_srt != nxt) | (iota16 == 15)
    return k_srt, perm, rank, last


def _sc1_body(slots_hbm, hists_hbm, slots_v, hist_v, ktmp):
    w = _wid()
    pltpu.sync_copy(slots_hbm.at[pl.ds(pl.multiple_of(w * PT, 128), PT)], slots_v)
    _zero_i32(hist_v, NB_PAD // 16)
    iota16 = lax.iota(jnp.int32, 16)

    def body(i, _):
        v = slots_v[pl.ds(i * 16, 16)] >> 10
        k_srt, _, rank, last = _sortmeta(v, ktmp, iota16)
        plsc.addupdate_scatter(hist_v, [k_srt], rank + 1, mask=last)
        return 0
    lax.fori_loop(0, PT // 16, body, 0)
    pltpu.sync_copy(hist_v, hists_hbm.at[w])


def _sc1(slots):
    f = pl.kernel(
        _sc1_body,
        out_type=jax.ShapeDtypeStruct((T, NB_PAD), jnp.int32),
        mesh=_mesh(),
        compiler_params=_SC_PARAMS,
        scratch_types=[
            pltpu.VMEM((PT,), jnp.int32),
            pltpu.VMEM((NB_PAD,), jnp.int32),
            pltpu.VMEM((16,), jnp.int32),
        ],
    )
    return f(slots)


def _accum_hists(hists_hbm, rowbuf, tot, cbelow, w):
    _zero_i32(tot, NB_PAD // 16)
    if cbelow is not None:
        _zero_i32(cbelow, NB_PAD // 16)

    def outer(wp, _):
        pltpu.sync_copy(hists_hbm.at[wp], rowbuf)

        def inner(i, _):
            sl = pl.ds(i * 16, 16)
            r = rowbuf[sl]
            tot[sl] = tot[sl] + r
            if cbelow is not None:
                use = (wp < w).astype(jnp.int32)
                cbelow[sl] = cbelow[sl] + r * use
            return 0
        lax.fori_loop(0, NB_PAD // 16, inner, 0)
        return 0
    lax.fori_loop(0, T, outer, 0)


def _group_walk(tot, binoff, meta_own, w):
    iota16 = lax.iota(jnp.int32, 16)

    def body(g, run):
        t = g // NGRP
        s = g - t * NGRP
        b0 = t * RPT + 2 * s
        tv = tot[pl.ds(b0, 16)]
        t0 = tv[0]
        t1 = tv[1]
        gsum = t0 + t1
        if binoff is not None:
            vals = run + jnp.where(iota16 == 1, t0, 0)
            plsc.store_scatter(binoff, [b0 + iota16], vals, mask=iota16 < 2)
        if meta_own is not None:
            m = (t == w) & (iota16 < 2)
            vals = jnp.where(iota16 == 0, run, gsum)
            plsc.store_scatter(meta_own, [s + iota16 * 32], vals, mask=m)
        return run + ((gsum + 7) & (-8))

    return lax.fori_loop(0, T * NGRP, body, jnp.int32(0))


_SPANS = [1024] * 9 + [256]


def _sc2_body(slots_hbm, h_hbm, hists_hbm,
              hsort_hbm, ssort_hbm,
              slots_v, rowbuf, tot, cbelow, binoff, off_v,
              hbufA, hbufB, tailbuf, posbuf, ktmp,
              sem_inA, sem_inB, sem_sc):
    w = _wid()
    pltpu.sync_copy(slots_hbm.at[pl.ds(pl.multiple_of(w * PT, 128), PT)], slots_v)
    _accum_hists(hists_hbm, rowbuf, tot, cbelow, w)
    run_final = _group_walk(tot, binoff, None, w)
    binoff[pl.ds(TRASH, 16)] = jnp.broadcast_to(run_final, (16,))

    def vadd(i, _):
        sl = pl.ds(i * 16, 16)
        off_v[sl] = binoff[sl] + cbelow[sl]
        return 0
    lax.fori_loop(0, NB_PAD // 16, vadd, 0)

    iota16 = lax.iota(jnp.int32, 16)

    nspans = len(_SPANS)
    hbufs = [hbufA, hbufB]
    sems = [sem_inA, sem_inB]

    def issue_in(s):
        size = _SPANS[s]
        buf = tailbuf if size == 256 else hbufs[s & 1]
        return pltpu.async_copy(
            h_hbm.at[pl.ds(w * PT + s * 1024, size)], buf,
            sems[s & 1])

    in_h = {0: issue_in(0)}
    scat = {}
    for s in range(nspans):
        size = _SPANS[s]
        buf = tailbuf if size == 256 else hbufs[s & 1]
        in_h[s].wait()
        if s - 1 >= 0:
            for cp in scat[s - 1]:
                cp.wait()
        if s + 1 < nspans:
            in_h[s + 1] = issue_in(s + 1)

        base = s * 1024

        def fill(kk, _, base=base, s=s):
            row = jnp.full((16,), (s & 1) * 8 + kk, jnp.int32)
            for g in range(8):
                vslot = slots_v[pl.ds(base + kk * 128 + g * 16, 16)]
                vbin = vslot >> 10
                k_srt, perm, rank, last = _sortmeta(vbin, ktmp, iota16)
                pos = plsc.load_gather(off_v, [k_srt]) + rank
                plsc.store_scatter(posbuf, [row, g * 16 + perm], pos)
                plsc.addupdate_scatter(off_v, [k_srt], rank + 1, mask=last)
            return 0
        lax.fori_loop(0, size // 128, fill, 0)

        cps = []
        for k in range(size // 128):
            prow = posbuf.at[(s & 1) * 8 + k]
            cps.append(pltpu.async_copy(
                buf.at[pl.ds(k * 128, 128)], hsort_hbm.at[prow], sem_sc))
            cps.append(pltpu.async_copy(
                slots_v.at[pl.ds(base + k * 128, 128)], ssort_hbm.at[prow],
                sem_sc))
        scat[s] = cps
    for cp in scat[nspans - 1]:
        cp.wait()


def _sc2(slots, h, hists):
    f = pl.kernel(
        _sc2_body,
        out_type=(
            jax.ShapeDtypeStruct((NR, C_OUT), jnp.float32),
            jax.ShapeDtypeStruct((NR,), jnp.int32),
        ),
        mesh=_mesh(),
        compiler_params=_SC_PARAMS,
        scratch_types=[
            pltpu.VMEM((PT,), jnp.int32),
            pltpu.VMEM((NB_PAD,), jnp.int32),
            pltpu.VMEM((NB_PAD,), jnp.int32),
            pltpu.VMEM((NB_PAD,), jnp.int32),
            pltpu.VMEM((NB_PAD,), jnp.int32),
            pltpu.VMEM((NB_PAD,), jnp.int32),
            pltpu.VMEM((1024, C_OUT), jnp.float32),
            pltpu.VMEM((1024, C_OUT), jnp.float32),
            pltpu.VMEM((256, C_OUT), jnp.float32),
            pltpu.VMEM((16, 128), jnp.int32),
            pltpu.VMEM((16,), jnp.int32),
            pltpu.SemaphoreType.DMA,
            pltpu.SemaphoreType.DMA,
            pltpu.SemaphoreType.DMA,
        ],
    )
    return f(slots, h, hists)


def _sc3_body(ssort_hbm, hsortf_hbm, hists_hbm, pooled_hbm,
              rowbuf, tot, meta_own, slabs, sbuf, hbuf,
              semfA, semfB):
    w = _wid()
    _accum_hists(hists_hbm, rowbuf, tot, None, w)
    _group_walk(tot, None, meta_own, w)
    semf = [semfA, semfB]
    flush = {}

    for s in range(NGRP):
        par = s & 1
        if s - 2 >= 0:
            flush[s - 2].wait()
        def zbody(i, _, par=par):
            slabs[par, pl.ds(i * 16, 16)] = jnp.zeros((16,), jnp.float32)
            return 0
        lax.fori_loop(0, SLAB_W // 16, zbody, 0)

        start = meta_own[pl.ds(s, 16)][0]
        ln = meta_own[pl.ds(32 + s, 16)][0]
        b0 = (w * RPT + 2 * s) * 1024
        nch = (ln + 1023) >> 10

        def chunk(c, _, par=par, start=start, ln=ln, b0=b0):
            cs = pl.multiple_of(start + c * 1024, 8)
            pltpu.sync_copy(ssort_hbm.at[pl.ds(cs, 1024)],
                            sbuf.at[pl.ds(0, 1024)])
            pltpu.sync_copy(hsortf_hbm.at[pl.ds(pl.multiple_of(cs * C_OUT, 256), 1024 * C_OUT)],
                            hbuf)
            cnt = jnp.minimum(jnp.int32(1024), ln - c * 1024)

            def point(j, _):
                sl = sbuf[pl.ds(j, 16)][0]
                addr = ((sl - b0) >> 10) * ROW_W + (sl & 1023) * C_OUT
                d0 = pl.ds(addr, 16)
                d1 = pl.ds(addr + 16, 16)
                h0 = hbuf[pl.ds(j * C_OUT, 16)]
                h1 = hbuf[pl.ds(j * C_OUT + 16, 16)]
                slabs[par, d0] = jnp.maximum(slabs[par, d0], h0)
                slabs[par, d1] = jnp.maximum(slabs[par, d1], h1)
                return 0
            lax.fori_loop(0, cnt, point, 0)
            return 0
        lax.fori_loop(0, nch, chunk, 0)

        dst = pooled_hbm.at[pl.ds(pl.multiple_of((w * RPT + 2 * s) * ROW_W, 256), SLAB_W)]
        flush[s] = pltpu.async_copy(slabs.at[par], dst, semf[par])

    flush[NGRP - 2].wait()
    flush[NGRP - 1].wait()


def _sc3(ssort, hsortf, hists):
    f = pl.kernel(
        _sc3_body,
        out_type=jax.ShapeDtypeStruct((NBIN * ROW_W,), jnp.float32),
        mesh=_mesh(),
        compiler_params=_SC_PARAMS,
        scratch_types=[
            pltpu.VMEM((NB_PAD,), jnp.int32),
            pltpu.VMEM((NB_PAD,), jnp.int32),
            pltpu.VMEM((80,), jnp.int32),
            pltpu.VMEM((2, SLAB_W), jnp.float32),
            pltpu.VMEM((1040,), jnp.int32),
            pltpu.VMEM((1024 * C_OUT,), jnp.float32),
            pltpu.SemaphoreType.DMA,
            pltpu.SemaphoreType.DMA,
        ],
    )
    return f(ssort, hsortf, hists)


_TR_BLK = 11264


def _tr_body(i_ref, id_ref, o_ref):
    x = i_ref[0]
    o_ref[0] = lax.dot_general(
        id_ref[...], x, (((1,), (1,)), ((), ())),
        preferred_element_type=jnp.float32,
        precision=lax.Precision.HIGHEST)


def _stage_d(pooled3):
    nblk = (H * W_GRID) // _TR_BLK
    ident = jnp.eye(C_OUT, dtype=jnp.float32)
    return pl.pallas_call(
        _tr_body,
        grid=(2, nblk),
        in_specs=[
            pl.BlockSpec((1, _TR_BLK, C_OUT), lambda b, r: (b, r, 0)),
            pl.BlockSpec((C_OUT, C_OUT), lambda b, r: (0, 0)),
        ],
        out_specs=pl.BlockSpec((1, C_OUT, _TR_BLK), lambda b, r: (b, 0, r)),
        out_shape=jax.ShapeDtypeStruct((2, C_OUT, H * W_GRID), jnp.float32),
    )(pooled3, ident)


def kernel(points, Wm, b, gamma, beta, mean, var):
    B, N, C = points.shape
    n_tot = B * N
    scale = gamma / jnp.sqrt(var + 1e-5)
    wm = Wm * scale[None, :]
    bb = ((b - mean) * scale + beta)[None, :]
    pts_flat = points.reshape(n_tot, C)
    pts_flat = jnp.pad(pts_flat, ((0, N_PAD - n_tot), (0, 8 - C)),
                       constant_values=-1e9)
    bo = jnp.where(jnp.arange(N_PAD, dtype=jnp.int32) >= N,
                   jnp.int32(H), jnp.int32(0))
    h, slots = _stage_a(pts_flat, bo, wm, bb)
    hists = _sc1(slots)
    hsort, ssort = _sc2(slots, h, hists)
    pooled = _sc3(ssort, hsort.reshape(-1), hists)
    out = _stage_d(pooled.reshape(B, H * W_GRID, C_OUT))
    return out.reshape(B, C_OUT, H, W_GRID)


# speedup vs baseline: 1.0625x; 1.0625x over previous
import functools

import jax
import jax.numpy as jnp
from jax import lax
from jax.experimental import pallas as pl
from jax.experimental.pallas import tpu as pltpu
from jax.experimental.pallas import tpu_sc as plsc

PC_RANGE = (0.0, -40.0, -3.0, 70.4, 40.0, 1.0)
PILLAR = 0.1
H = 800
W_GRID = 704
C_OUT = 32

NBIN = 1600
TRASH = NBIN
NB_PAD = 1664
T = 32
PT = 9472
N_PAD = PT * T
RPT = 50
NGRP = 25
ROW_W = W_GRID * C_OUT
SLAB_W = 2 * ROW_W
NR = 310272

_BLK = 2048

_SC_PARAMS = pltpu.CompilerParams(needs_layout_passes=False,
                                  use_tc_tiling_on_sc=False)


def _pfe_body(pts_ref, bo_ref, wm_ref, bb_ref, h_ref, slot_ref):
    pts = pts_ref[...]
    x = pts[:, 0]
    y = pts[:, 1]
    cx = jnp.floor((x - PC_RANGE[0]) / PILLAR).astype(jnp.int32)
    cy = jnp.floor((y - PC_RANGE[1]) / PILLAR).astype(jnp.int32)
    mask = (cx >= 0) & (cx < W_GRID) & (cy >= 0) & (cy < H)
    cxc = jnp.clip(cx, 0, W_GRID - 1)
    cyc = jnp.clip(cy, 0, H - 1)
    center_x = (cxc.astype(jnp.float32) + 0.5) * PILLAR + PC_RANGE[0]
    center_y = (cyc.astype(jnp.float32) + 0.5) * PILLAR + PC_RANGE[1]
    feats = jnp.concatenate(
        [pts[:, :5], (x - center_x)[:, None], (y - center_y)[:, None]], axis=1
    )
    h = jnp.dot(feats, wm_ref[...], preferred_element_type=jnp.float32) + bb_ref[...]
    h_ref[...] = jnp.maximum(h, 0.0)
    slot = (bo_ref[...] + cyc) * 1024 + cxc
    slot_ref[...] = jnp.where(mask, slot, jnp.int32(TRASH * 1024))


def _stage_a(pts_flat, bo, wm, bb):
    return pl.pallas_call(
        _pfe_body,
        grid=(N_PAD // _BLK,),
        in_specs=[
            pl.BlockSpec((_BLK, 8), lambda i: (i, 0)),
            pl.BlockSpec((_BLK,), lambda i: (i,)),
            pl.BlockSpec((7, C_OUT), lambda i: (0, 0)),
            pl.BlockSpec((1, C_OUT), lambda i: (0, 0)),
        ],
        out_specs=[
            pl.BlockSpec((_BLK, C_OUT), lambda i: (i, 0)),
            pl.BlockSpec((_BLK,), lambda i: (i,)),
        ],
        out_shape=[
            jax.ShapeDtypeStruct((N_PAD, C_OUT), jnp.float32),
            jax.ShapeDtypeStruct((N_PAD,), jnp.int32),
        ],
    )(pts_flat, bo, wm, bb)


def _mesh():
    return plsc.VectorSubcoreMesh(core_axis_name="c", subcore_axis_name="s")


def _wid():
    return lax.axis_index("s") * 2 + lax.axis_index("c")


def _zero_i32(ref, nvec):
    def body(i, _):
        ref[pl.ds(i * 16, 16)] = jnp.zeros((16,), jnp.int32)
        return 0
    lax.fori_loop(0, nvec, body, 0)


def _sortmeta(vbin, ktmp, iota16):
    k_srt, perm = plsc.sort_key_val(vbin, iota16)
    ktmp[...] = k_srt
    prev = plsc.load_gather(ktmp, [jnp.maximum(iota16 - 1, 0)])
    newseg = (k_srt != prev) | (iota16 == 0)
    segstart = plsc.cummax(jnp.where(newseg, iota16, 0))
    rank = iota16 - segstart
    nxt = plsc.load_gather(ktmp, [jnp.minimum(iota16 + 1, 15)])
    last = (k_srt != nxt) | (iota16 == 15)
    return k_srt, perm, rank, last


def _sc1_body(slots_hbm, hists_hbm, slots_v, hist_v, ktmp):
    w = _wid()
    pltpu.sync_copy(slots_hbm.at[pl.ds(pl.multiple_of(w * PT, 128), PT)], slots_v)
    _zero_i32(hist_v, NB_PAD // 16)
    iota16 = lax.iota(jnp.int32, 16)

    def body(i, _):
        v = slots_v[pl.ds(i * 16, 16)] >> 10
        k_srt, _, rank, last = _sortmeta(v, ktmp, iota16)
        plsc.addupdate_scatter(hist_v, [k_srt], rank + 1, mask=last)
        return 0
    lax.fori_loop(0, PT // 16, body, 0)
    pltpu.sync_copy(hist_v, hists_hbm.at[w])


def _sc1(slots):
    f = pl.kernel(
        _sc1_body,
        out_type=jax.ShapeDtypeStruct((T, NB_PAD), jnp.int32),
        mesh=_mesh(),
        compiler_params=_SC_PARAMS,
        scratch_types=[
            pltpu.VMEM((PT,), jnp.int32),
            pltpu.VMEM((NB_PAD,), jnp.int32),
            pltpu.VMEM((16,), jnp.int32),
        ],
    )
    return f(slots)


def _accum_hists(hists_hbm, rowbuf, tot, cbelow, w):
    _zero_i32(tot, NB_PAD // 16)
    if cbelow is not None:
        _zero_i32(cbelow, NB_PAD // 16)

    def outer(wp, _):
        pltpu.sync_copy(hists_hbm.at[wp], rowbuf)

        def inner(i, _):
            sl = pl.ds(i * 16, 16)
            r = rowbuf[sl]
            tot[sl] = tot[sl] + r
            if cbelow is not None:
                use = (wp < w).astype(jnp.int32)
                cbelow[sl] = cbelow[sl] + r * use
            return 0
        lax.fori_loop(0, NB_PAD // 16, inner, 0)
        return 0
    lax.fori_loop(0, T, outer, 0)


def _group_walk(tot, binoff, meta_own, w):
    iota16 = lax.iota(jnp.int32, 16)

    def body(g, run):
        t = g // NGRP
        s = g - t * NGRP
        b0 = t * RPT + 2 * s
        tv = tot[pl.ds(b0, 16)]
        t0 = tv[0]
        t1 = tv[1]
        gsum = t0 + t1
        if binoff is not None:
            vals = run + jnp.where(iota16 == 1, t0, 0)
            plsc.store_scatter(binoff, [b0 + iota16], vals, mask=iota16 < 2)
        if meta_own is not None:
            m = (t == w) & (iota16 < 2)
            vals = jnp.where(iota16 == 0, run, gsum)
            plsc.store_scatter(meta_own, [s + iota16 * 32], vals, mask=m)
        return run + ((gsum + 7) & (-8))

    return lax.fori_loop(0, T * NGRP, body, jnp.int32(0))


_SPANS = [1024] * 9 + [256]


def _sc2_body(slots_hbm, h_hbm, hists_hbm,
              hsort_hbm, ssort_hbm,
              slots_v, rowbuf, tot, cbelow, binoff, off_v,
              hbufA, hbufB, tailbuf, posbuf, ktmp,
              sem_inA, sem_inB, sem_sc):
    w = _wid()
    pltpu.sync_copy(slots_hbm.at[pl.ds(pl.multiple_of(w * PT, 128), PT)], slots_v)
    _accum_hists(hists_hbm, rowbuf, tot, cbelow, w)
    run_final = _group_walk(tot, binoff, None, w)
    binoff[pl.ds(TRASH, 16)] = jnp.broadcast_to(run_final, (16,))

    def vadd(i, _):
        sl = pl.ds(i * 16, 16)
        off_v[sl] = binoff[sl] + cbelow[sl]
        return 0
    lax.fori_loop(0, NB_PAD // 16, vadd, 0)

    iota16 = lax.iota(jnp.int32, 16)

    nspans = len(_SPANS)
    hbufs = [hbufA, hbufB]
    sems = [sem_inA, sem_inB]

    def issue_in(s):
        size = _SPANS[s]
        buf = tailbuf if size == 256 else hbufs[s & 1]
        return pltpu.async_copy(
            h_hbm.at[pl.ds(w * PT + s * 1024, size)], buf,
            sems[s & 1])

    in_h = {0: issue_in(0)}
    scat = {}
    for s in range(nspans):
        size = _SPANS[s]
        buf = tailbuf if size == 256 else hbufs[s & 1]
        in_h[s].wait()
        if s - 1 >= 0:
            for cp in scat[s - 1]:
                cp.wait()
        if s + 1 < nspans:
            in_h[s + 1] = issue_in(s + 1)

        base = s * 1024

        def fill(kk, _, base=base, s=s):
            row = jnp.full((16,), (s & 1) * 8 + kk, jnp.int32)
            for g in range(8):
                vslot = slots_v[pl.ds(base + kk * 128 + g * 16, 16)]
                vbin = vslot >> 10
                k_srt, perm, rank, last = _sortmeta(vbin, ktmp, iota16)
                pos = plsc.load_gather(off_v, [k_srt]) + rank
                plsc.store_scatter(posbuf, [row, g * 16 + perm], pos)
                plsc.addupdate_scatter(off_v, [k_srt], rank + 1, mask=last)
            return 0
        lax.fori_loop(0, size // 128, fill, 0)

        cps = []
        for k in range(size // 128):
            prow = posbuf.at[(s & 1) * 8 + k]
            cps.append(pltpu.async_copy(
                buf.at[pl.ds(k * 128, 128)], hsort_hbm.at[prow], sem_sc))
            cps.append(pltpu.async_copy(
                slots_v.at[pl.ds(base + k * 128, 128)], ssort_hbm.at[prow],
                sem_sc))
        scat[s] = cps
    for cp in scat[nspans - 1]:
        cp.wait()


def _sc2(slots, h, hists):
    f = pl.kernel(
        _sc2_body,
        out_type=(
            jax.ShapeDtypeStruct((NR, C_OUT), jnp.float32),
            jax.ShapeDtypeStruct((NR,), jnp.int32),
        ),
        mesh=_mesh(),
        compiler_params=_SC_PARAMS,
        scratch_types=[
            pltpu.VMEM((PT,), jnp.int32),
            pltpu.VMEM((NB_PAD,), jnp.int32),
            pltpu.VMEM((NB_PAD,), jnp.int32),
            pltpu.VMEM((NB_PAD,), jnp.int32),
            pltpu.VMEM((NB_PAD,), jnp.int32),
            pltpu.VMEM((NB_PAD,), jnp.int32),
            pltpu.VMEM((1024, C_OUT), jnp.float32),
            pltpu.VMEM((1024, C_OUT), jnp.float32),
            pltpu.VMEM((256, C_OUT), jnp.float32),
            pltpu.VMEM((16, 128), jnp.int32),
            pltpu.VMEM((16,), jnp.int32),
            pltpu.SemaphoreType.DMA,
            pltpu.SemaphoreType.DMA,
            pltpu.SemaphoreType.DMA,
        ],
    )
    return f(slots, h, hists)


def _sc3_body(ssort_hbm, hsort_hbm, hists_hbm, pooled_hbm,
              rowbuf, tot, meta_own, slabA, slabB, sbuf, hbuf,
              semfA, semfB):
    w = _wid()
    _accum_hists(hists_hbm, rowbuf, tot, None, w)
    _group_walk(tot, None, meta_own, w)
    semf = [semfA, semfB]
    slabs = [slabA, slabB]
    flush = {}
    zero16 = jnp.zeros((16,), jnp.float32)

    for s in range(NGRP):
        par = s & 1
        slab = slabs[par]
        if s - 2 >= 0:
            flush[s - 2].wait()

        def zbody(i, _, slab=slab):
            slab[i, pl.ds(0, 16)] = zero16
            slab[i, pl.ds(16, 16)] = zero16
            return 0
        lax.fori_loop(0, 2 * W_GRID, zbody, 0)

        start = meta_own[pl.ds(s, 16)][0]
        ln = meta_own[pl.ds(32 + s, 16)][0]
        b0 = (w * RPT + 2 * s) * 1024
        nch = (ln + 1023) >> 10

        def chunk(c, _, slab=slab, start=start, ln=ln, b0=b0):
            cs = pl.multiple_of(start + c * 1024, 8)
            pltpu.sync_copy(ssort_hbm.at[pl.ds(cs, 1024)],
                            sbuf.at[pl.ds(0, 1024)])
            pltpu.sync_copy(hsort_hbm.at[pl.ds(cs, 1024)], hbuf)
            cnt = jnp.minimum(jnp.int32(1024), ln - c * 1024)

            def point(j, _):
                sl = sbuf[pl.ds(j, 16)][0]
                prow = ((sl - b0) >> 10) * W_GRID + (sl & 1023)
                d0 = pl.ds(0, 16)
                d1 = pl.ds(16, 16)
                slab[prow, d0] = jnp.maximum(slab[prow, d0], hbuf[j, d0])
                slab[prow, d1] = jnp.maximum(slab[prow, d1], hbuf[j, d1])
                return 0
            lax.fori_loop(0, cnt, point, 0)
            return 0
        lax.fori_loop(0, nch, chunk, 0)

        dst = pooled_hbm.at[pl.ds((w * RPT + 2 * s) * W_GRID, 2 * W_GRID)]
        flush[s] = pltpu.async_copy(slab, dst, semf[par])

    flush[NGRP - 2].wait()
    flush[NGRP - 1].wait()


def _sc3(ssort, hsort, hists):
    f = pl.kernel(
        _sc3_body,
        out_type=jax.ShapeDtypeStruct((2 * H * W_GRID, C_OUT), jnp.float32),
        mesh=_mesh(),
        compiler_params=_SC_PARAMS,
        scratch_types=[
            pltpu.VMEM((NB_PAD,), jnp.int32),
            pltpu.VMEM((NB_PAD,), jnp.int32),
            pltpu.VMEM((80,), jnp.int32),
            pltpu.VMEM((2 * W_GRID, C_OUT), jnp.float32),
            pltpu.VMEM((2 * W_GRID, C_OUT), jnp.float32),
            pltpu.VMEM((1040,), jnp.int32),
            pltpu.VMEM((1024, C_OUT), jnp.float32),
            pltpu.SemaphoreType.DMA,
            pltpu.SemaphoreType.DMA,
        ],
    )
    return f(ssort, hsort, hists)


_TR_HB = 8


def _tr_body(i_ref, id_ref, o_ref):
    x = i_ref[...]
    ident = id_ref[...]
    for r in range(_TR_HB):
        o_ref[0, :, r, :] = lax.dot_general(
            ident, x[r * W_GRID:(r + 1) * W_GRID, :], (((1,), (1,)), ((), ())),
            preferred_element_type=jnp.float32,
            precision=lax.Precision.HIGHEST)


def _stage_d(pooled3):
    nblk = H // _TR_HB
    ident = jnp.eye(C_OUT, dtype=jnp.float32)
    return pl.pallas_call(
        _tr_body,
        grid=(2, nblk),
        in_specs=[
            pl.BlockSpec((_TR_HB * W_GRID, C_OUT),
                         lambda b, r: (b * (H // _TR_HB) + r, 0)),
            pl.BlockSpec((C_OUT, C_OUT), lambda b, r: (0, 0)),
        ],
        out_specs=pl.BlockSpec((1, C_OUT, _TR_HB, W_GRID),
                               lambda b, r: (b, 0, r, 0)),
        out_shape=jax.ShapeDtypeStruct((2, C_OUT, H, W_GRID), jnp.float32),
    )(pooled3, ident)


def kernel(points, Wm, b, gamma, beta, mean, var):
    B, N, C = points.shape
    n_tot = B * N
    scale = gamma / jnp.sqrt(var + 1e-5)
    wm = Wm * scale[None, :]
    bb = ((b - mean) * scale + beta)[None, :]
    pts_flat = points.reshape(n_tot, C)
    pts_flat = jnp.pad(pts_flat, ((0, N_PAD - n_tot), (0, 8 - C)),
                       constant_values=-1e9)
    bo = jnp.where(jnp.arange(N_PAD, dtype=jnp.int32) >= N,
                   jnp.int32(H), jnp.int32(0))
    h, slots = _stage_a(pts_flat, bo, wm, bb)
    hists = _sc1(slots)
    hsort, ssort = _sc2(slots, h, hists)
    pooled = _sc3(ssort, hsort, hists)
    return _stage_d(pooled)


# speedup vs baseline: 1.1682x; 1.0995x over previous
import functools

import jax
import jax.numpy as jnp
from jax import lax
from jax.experimental import pallas as pl
from jax.experimental.pallas import tpu as pltpu
from jax.experimental.pallas import tpu_sc as plsc

PC_RANGE = (0.0, -40.0, -3.0, 70.4, 40.0, 1.0)
PILLAR = 0.1
H = 800
W_GRID = 704
C_OUT = 32

NBIN = 1600
TRASH = NBIN
NB_PAD = 1664
T = 32
PT = 9472
N_PAD = PT * T
RPT = 50
NGRP = 25
ROW_W = W_GRID * C_OUT
SLAB_W = 2 * ROW_W
NR = 310272

_BLK = 2048

_SC_PARAMS = pltpu.CompilerParams(needs_layout_passes=False,
                                  use_tc_tiling_on_sc=False)


def _pfe_body(pts_ref, bo_ref, wm_ref, bb_ref, h_ref, slot_ref):
    pts = pts_ref[...]
    x = pts[:, 0]
    y = pts[:, 1]
    cx = jnp.floor((x - PC_RANGE[0]) / PILLAR).astype(jnp.int32)
    cy = jnp.floor((y - PC_RANGE[1]) / PILLAR).astype(jnp.int32)
    mask = (cx >= 0) & (cx < W_GRID) & (cy >= 0) & (cy < H)
    cxc = jnp.clip(cx, 0, W_GRID - 1)
    cyc = jnp.clip(cy, 0, H - 1)
    center_x = (cxc.astype(jnp.float32) + 0.5) * PILLAR + PC_RANGE[0]
    center_y = (cyc.astype(jnp.float32) + 0.5) * PILLAR + PC_RANGE[1]
    wm = wm_ref[...]
    h = bb_ref[...] + pts[:, 0:1] * wm[0:1, :]
    for k in range(1, 5):
        h = h + pts[:, k:k + 1] * wm[k:k + 1, :]
    h = h + (x - center_x)[:, None] * wm[5:6, :]
    h = h + (y - center_y)[:, None] * wm[6:7, :]
    h_ref[...] = jnp.maximum(h, 0.0)
    slot = (bo_ref[...] + cyc) * 1024 + cxc
    slot_ref[...] = jnp.where(mask, slot, jnp.int32(TRASH * 1024))


def _stage_a(pts_flat, bo, wm, bb):
    return pl.pallas_call(
        _pfe_body,
        grid=(N_PAD // _BLK,),
        in_specs=[
            pl.BlockSpec((_BLK, 8), lambda i: (i, 0)),
            pl.BlockSpec((_BLK,), lambda i: (i,)),
            pl.BlockSpec((7, C_OUT), lambda i: (0, 0)),
            pl.BlockSpec((1, C_OUT), lambda i: (0, 0)),
        ],
        out_specs=[
            pl.BlockSpec((_BLK, C_OUT), lambda i: (i, 0)),
            pl.BlockSpec((_BLK,), lambda i: (i,)),
        ],
        out_shape=[
            jax.ShapeDtypeStruct((N_PAD, C_OUT), jnp.float32),
            jax.ShapeDtypeStruct((N_PAD,), jnp.int32),
        ],
    )(pts_flat, bo, wm, bb)


def _mesh():
    return plsc.VectorSubcoreMesh(core_axis_name="c", subcore_axis_name="s")


def _wid():
    return lax.axis_index("s") * 2 + lax.axis_index("c")


def _zero_i32(ref, nvec):
    def body(i, _):
        ref[pl.ds(i * 16, 16)] = jnp.zeros((16,), jnp.int32)
        return 0
    lax.fori_loop(0, nvec, body, 0)


def _sortmeta(vbin, ktmp, iota16):
    k_srt, perm = plsc.sort_key_val(vbin, iota16)
    ktmp[...] = k_srt
    prev = plsc.load_gather(ktmp, [jnp.maximum(iota16 - 1, 0)])
    newseg = (k_srt != prev) | (iota16 == 0)
    segstart = plsc.cummax(jnp.where(newseg, iota16, 0))
    rank = iota16 - segstart
    nxt = plsc.load_gather(ktmp, [jnp.minimum(iota16 + 1, 15)])
    last = (k_srt != nxt) | (iota16 == 15)
    return k_srt, perm, rank, last


def _sc1_body(slots_hbm, hists_hbm, slots_v, hist_v, ktmp):
    w = _wid()
    pltpu.sync_copy(slots_hbm.at[pl.ds(pl.multiple_of(w * PT, 128), PT)], slots_v)
    _zero_i32(hist_v, NB_PAD // 16)
    iota16 = lax.iota(jnp.int32, 16)

    def body(i, _):
        v = slots_v[pl.ds(i * 16, 16)] >> 10
        k_srt, _, rank, last = _sortmeta(v, ktmp, iota16)
        plsc.addupdate_scatter(hist_v, [k_srt], rank + 1, mask=last)
        return 0
    lax.fori_loop(0, PT // 16, body, 0)
    pltpu.sync_copy(hist_v, hists_hbm.at[w])


def _sc1(slots):
    f = pl.kernel(
        _sc1_body,
        out_type=jax.ShapeDtypeStruct((T, NB_PAD), jnp.int32),
        mesh=_mesh(),
        compiler_params=_SC_PARAMS,
        scratch_types=[
            pltpu.VMEM((PT,), jnp.int32),
            pltpu.VMEM((NB_PAD,), jnp.int32),
            pltpu.VMEM((16,), jnp.int32),
        ],
    )
    return f(slots)


def _accum_hists(hists_hbm, rowbuf, tot, cbelow, w):
    _zero_i32(tot, NB_PAD // 16)
    if cbelow is not None:
        _zero_i32(cbelow, NB_PAD // 16)

    def outer(wp, _):
        pltpu.sync_copy(hists_hbm.at[wp], rowbuf)

        def inner(i, _):
            sl = pl.ds(i * 16, 16)
            r = rowbuf[sl]
            tot[sl] = tot[sl] + r
            if cbelow is not None:
                use = (wp < w).astype(jnp.int32)
                cbelow[sl] = cbelow[sl] + r * use
            return 0
        lax.fori_loop(0, NB_PAD // 16, inner, 0)
        return 0
    lax.fori_loop(0, T, outer, 0)


def _group_walk(tot, binoff, meta_own, w):
    iota16 = lax.iota(jnp.int32, 16)

    def body(g, run):
        t = g // NGRP
        s = g - t * NGRP
        b0 = t * RPT + 2 * s
        tv = tot[pl.ds(b0, 16)]
        t0 = tv[0]
        t1 = tv[1]
        gsum = t0 + t1
        if binoff is not None:
            vals = run + jnp.where(iota16 == 1, t0, 0)
            plsc.store_scatter(binoff, [b0 + iota16], vals, mask=iota16 < 2)
        if meta_own is not None:
            m = (t == w) & (iota16 < 2)
            vals = jnp.where(iota16 == 0, run, gsum)
            plsc.store_scatter(meta_own, [s + iota16 * 32], vals, mask=m)
        return run + ((gsum + 7) & (-8))

    return lax.fori_loop(0, T * NGRP, body, jnp.int32(0))


_SPANS = [1024] * 9 + [256]


def _sc2_body(slots_hbm, h_hbm, hists_hbm,
              hsort_hbm, ssort_hbm,
              slots_v, rowbuf, tot, cbelow, binoff, off_v,
              hbufA, hbufB, tailbuf, posbuf, ktmp,
              sem_inA, sem_inB, sem_sc):
    w = _wid()
    pltpu.sync_copy(slots_hbm.at[pl.ds(pl.multiple_of(w * PT, 128), PT)], slots_v)
    _accum_hists(hists_hbm, rowbuf, tot, cbelow, w)
    run_final = _group_walk(tot, binoff, None, w)
    binoff[pl.ds(TRASH, 16)] = jnp.broadcast_to(run_final, (16,))

    def vadd(i, _):
        sl = pl.ds(i * 16, 16)
        off_v[sl] = binoff[sl] + cbelow[sl]
        return 0
    lax.fori_loop(0, NB_PAD // 16, vadd, 0)

    iota16 = lax.iota(jnp.int32, 16)

    nspans = len(_SPANS)
    hbufs = [hbufA, hbufB]
    sems = [sem_inA, sem_inB]

    def issue_in(s):
        size = _SPANS[s]
        buf = tailbuf if size == 256 else hbufs[s & 1]
        return pltpu.async_copy(
            h_hbm.at[pl.ds(w * PT + s * 1024, size)], buf,
            sems[s & 1])

    in_h = {0: issue_in(0)}
    scat = {}
    for s in range(nspans):
        size = _SPANS[s]
        buf = tailbuf if size == 256 else hbufs[s & 1]
        in_h[s].wait()
        if s - 1 >= 0:
            for cp in scat[s - 1]:
                cp.wait()
        if s + 1 < nspans:
            in_h[s + 1] = issue_in(s + 1)

        base = s * 1024

        def fill(kk, _, base=base, s=s):
            row = jnp.full((16,), (s & 1) * 8 + kk, jnp.int32)
            for g in range(8):
                vslot = slots_v[pl.ds(base + kk * 128 + g * 16, 16)]
                vbin = vslot >> 10
                k_srt, perm, rank, last = _sortmeta(vbin, ktmp, iota16)
                pos = plsc.load_gather(off_v, [k_srt]) + rank
                plsc.store_scatter(posbuf, [row, g * 16 + perm], pos)
                plsc.addupdate_scatter(off_v, [k_srt], rank + 1, mask=last)
            return 0
        lax.fori_loop(0, size // 128, fill, 0)

        cps = []
        for k in range(size // 128):
            prow = posbuf.at[(s & 1) * 8 + k]
            cps.append(pltpu.async_copy(
                buf.at[pl.ds(k * 128, 128)], hsort_hbm.at[prow], sem_sc))
            cps.append(pltpu.async_copy(
                slots_v.at[pl.ds(base + k * 128, 128)], ssort_hbm.at[prow],
                sem_sc))
        scat[s] = cps
    for cp in scat[nspans - 1]:
        cp.wait()


def _sc2(slots, h, hists):
    f = pl.kernel(
        _sc2_body,
        out_type=(
            jax.ShapeDtypeStruct((NR, C_OUT), jnp.float32),
            jax.ShapeDtypeStruct((NR,), jnp.int32),
        ),
        mesh=_mesh(),
        compiler_params=_SC_PARAMS,
        scratch_types=[
            pltpu.VMEM((PT,), jnp.int32),
            pltpu.VMEM((NB_PAD,), jnp.int32),
            pltpu.VMEM((NB_PAD,), jnp.int32),
            pltpu.VMEM((NB_PAD,), jnp.int32),
            pltpu.VMEM((NB_PAD,), jnp.int32),
            pltpu.VMEM((NB_PAD,), jnp.int32),
            pltpu.VMEM((1024, C_OUT), jnp.float32),
            pltpu.VMEM((1024, C_OUT), jnp.float32),
            pltpu.VMEM((256, C_OUT), jnp.float32),
            pltpu.VMEM((16, 128), jnp.int32),
            pltpu.VMEM((16,), jnp.int32),
            pltpu.SemaphoreType.DMA,
            pltpu.SemaphoreType.DMA,
            pltpu.SemaphoreType.DMA,
        ],
    )
    return f(slots, h, hists)


def _sc3_body(ssort_hbm, hsort_hbm, hists_hbm, pooled_hbm,
              rowbuf, tot, meta_own, slabA, slabB, sbuf, hbuf,
              semfA, semfB):
    w = _wid()
    _accum_hists(hists_hbm, rowbuf, tot, None, w)
    _group_walk(tot, None, meta_own, w)
    semf = [semfA, semfB]
    slabs = [slabA, slabB]
    flush = {}
    zero16 = jnp.zeros((16,), jnp.float32)

    for s in range(NGRP):
        par = s & 1
        slab = slabs[par]
        if s - 2 >= 0:
            flush[s - 2].wait()

        def zbody(i, _, slab=slab):
            slab[i, pl.ds(0, 16)] = zero16
            slab[i, pl.ds(16, 16)] = zero16
            return 0
        lax.fori_loop(0, 2 * W_GRID, zbody, 0)

        start = meta_own[pl.ds(s, 16)][0]
        ln = meta_own[pl.ds(32 + s, 16)][0]
        b0 = (w * RPT + 2 * s) * 1024
        nch = (ln + 1023) >> 10

        def chunk(c, _, slab=slab, start=start, ln=ln, b0=b0):
            cs = pl.multiple_of(start + c * 1024, 8)
            pltpu.sync_copy(ssort_hbm.at[pl.ds(cs, 1024)],
                            sbuf.at[pl.ds(0, 1024)])
            pltpu.sync_copy(hsort_hbm.at[pl.ds(cs, 1024)], hbuf)
            cnt = jnp.minimum(jnp.int32(1024), ln - c * 1024)

            def point(j, _):
                sl = sbuf[pl.ds(j, 16)][0]
                prow = ((sl - b0) >> 10) * W_GRID + (sl & 1023)
                d0 = pl.ds(0, 16)
                d1 = pl.ds(16, 16)
                slab[prow, d0] = jnp.maximum(slab[prow, d0], hbuf[j, d0])
                slab[prow, d1] = jnp.maximum(slab[prow, d1], hbuf[j, d1])
                return 0
            lax.fori_loop(0, cnt, point, 0)
            return 0
        lax.fori_loop(0, nch, chunk, 0)

        dst = pooled_hbm.at[pl.ds((w * RPT + 2 * s) * W_GRID, 2 * W_GRID)]
        flush[s] = pltpu.async_copy(slab, dst, semf[par])

    flush[NGRP - 2].wait()
    flush[NGRP - 1].wait()


def _sc3(ssort, hsort, hists):
    f = pl.kernel(
        _sc3_body,
        out_type=jax.ShapeDtypeStruct((2 * H * W_GRID, C_OUT), jnp.float32),
        mesh=_mesh(),
        compiler_params=_SC_PARAMS,
        scratch_types=[
            pltpu.VMEM((NB_PAD,), jnp.int32),
            pltpu.VMEM((NB_PAD,), jnp.int32),
            pltpu.VMEM((80,), jnp.int32),
            pltpu.VMEM((2 * W_GRID, C_OUT), jnp.float32),
            pltpu.VMEM((2 * W_GRID, C_OUT), jnp.float32),
            pltpu.VMEM((1040,), jnp.int32),
            pltpu.VMEM((1024, C_OUT), jnp.float32),
            pltpu.SemaphoreType.DMA,
            pltpu.SemaphoreType.DMA,
        ],
    )
    return f(ssort, hsort, hists)


_TR_HB = 8


def _tr_body(i_ref, id_ref, o_ref):
    x = i_ref[...]
    ident = id_ref[...]
    for r in range(_TR_HB):
        o_ref[0, :, r, :] = lax.dot_general(
            ident, x[r * W_GRID:(r + 1) * W_GRID, :], (((1,), (1,)), ((), ())),
            preferred_element_type=jnp.float32,
            precision=lax.Precision.HIGHEST)


def _stage_d(pooled3):
    nblk = H // _TR_HB
    ident = jnp.eye(C_OUT, dtype=jnp.float32)
    return pl.pallas_call(
        _tr_body,
        grid=(2, nblk),
        in_specs=[
            pl.BlockSpec((_TR_HB * W_GRID, C_OUT),
                         lambda b, r: (b * (H // _TR_HB) + r, 0)),
            pl.BlockSpec((C_OUT, C_OUT), lambda b, r: (0, 0)),
        ],
        out_specs=pl.BlockSpec((1, C_OUT, _TR_HB, W_GRID),
                               lambda b, r: (b, 0, r, 0)),
        out_shape=jax.ShapeDtypeStruct((2, C_OUT, H, W_GRID), jnp.float32),
    )(pooled3, ident)


def kernel(points, Wm, b, gamma, beta, mean, var):
    B, N, C = points.shape
    n_tot = B * N
    scale = gamma / jnp.sqrt(var + 1e-5)
    wm = Wm * scale[None, :]
    bb = ((b - mean) * scale + beta)[None, :]
    pts_flat = points.reshape(n_tot, C)
    pts_flat = jnp.pad(pts_flat, ((0, N_PAD - n_tot), (0, 8 - C)),
                       constant_values=-1e9)
    bo = jnp.where(jnp.arange(N_PAD, dtype=jnp.int32) >= N,
                   jnp.int32(H), jnp.int32(0))
    h, slots = _stage_a(pts_flat, bo, wm, bb)
    hists = _sc1(slots)
    hsort, ssort = _sc2(slots, h, hists)
    pooled = _sc3(ssort, hsort, hists)
    return pooled.reshape(B, H, W_GRID, C_OUT).transpose(0, 3, 1, 2)
